# baseline (speedup 1.0000x reference)
import functools

import jax
import jax.numpy as jnp
from jax import lax
from jax.experimental import pallas as pl
from jax.experimental.pallas import tpu as pltpu
from jax.experimental.pallas import tpu_sc as plsc

N = 10000
E = 320000
D_IN = 128
D_H = 256
NC, NS = 2, 16
K = 128

OCT = 8
ST_T = 160
E_PT = ST_T * K
E_PAD = E_PT * NS
N_H = 10240
NA = N + 16
R0 = 632
R_LAST = N - (NS - 1) * R0

_mesh = plsc.VectorSubcoreMesh(
    core_axis_name="c", subcore_axis_name="s", num_cores=NC, num_subcores=NS)


_HR = N_H // 16
_NW = NC * NS


def _deg_body(col_hbm, deg_hbm, col_v, hist):
    c = lax.axis_index("c")
    s = lax.axis_index("s")
    w = c * NS + s
    e_pw = E_PAD // _NW
    z16 = jnp.zeros((16,), jnp.float32)

    def zero_hist(i, _):
        hist[i, :] = z16
        return 0
    lax.fori_loop(0, _HR, zero_hist, 0)

    pltpu.sync_copy(col_hbm.at[pl.ds(w * e_pw, e_pw)], col_v)
    ones = jnp.ones((16,), jnp.float32)

    def count(i, _):
        idx = col_v[pl.ds(i * 16, 16)]
        plsc.addupdate_scatter(hist, [idx >> 4, idx & 15], ones)
        return 0
    lax.fori_loop(0, e_pw // 16, count, 0)

    pltpu.sync_copy(hist, deg_hbm.at[pl.ds(w * _HR, _HR)])


_deg_call = functools.partial(
    pl.kernel, _deg_body,
    out_type=jax.ShapeDtypeStruct((_NW * _HR, 16), jnp.float32),
    mesh=_mesh,
    compiler_params=pltpu.CompilerParams(needs_layout_passes=False),
    scratch_types=[
        pltpu.VMEM((E_PAD // (NC * NS),), jnp.int32),
        pltpu.VMEM((_HR, 16), jnp.float32),
    ],
)()


def _scatter_body(y_hbm, g_hbm, c_hbm, out_hbm, gb0, gb1, cb0, cb1, r0, r1,
                  acc_sh, gs0, gs1):
    c = lax.axis_index("c")
    s = lax.axis_index("s")
    @pl.when(s < NS - 1)
    def _init_main():
        pltpu.sync_copy(y_hbm.at[pl.ds(c * N + s * R0, R0)],
                        acc_sh.at[pl.ds(s * R0, R0)])

    @pl.when(s == NS - 1)
    def _init_last():
        pltpu.sync_copy(y_hbm.at[pl.ds(c * N + s * R0, R_LAST)],
                        acc_sh.at[pl.ds(s * R0, R_LAST)])
    plsc.subcore_barrier()

    gbase = c * (E_PAD // K) + s * ST_T
    cbase = s * ST_T
    n_oct = ST_T // OCT

    def oct_body(o, _):
        pltpu.sync_copy(g_hbm.at[pl.ds(gbase + o * OCT, OCT)], gb0)
        pltpu.sync_copy(c_hbm.at[pl.ds(cbase + o * OCT, OCT)], cb0)
        for t in range(OCT):
            pltpu.async_copy(y_hbm.at[gb0.at[t]], r0, gs0).wait()
            pltpu.sync_copy(r0, acc_sh.at[cb0.at[t]], add=True)
        return 0
    lax.fori_loop(0, n_oct, oct_body, 0)

    plsc.subcore_barrier()

    @pl.when(s < NS - 1)
    def _out_main():
        pltpu.sync_copy(acc_sh.at[pl.ds(s * R0, R0)],
                        out_hbm.at[pl.ds(c * N + s * R0, R0)])

    @pl.when(s == NS - 1)
    def _out_last():
        pltpu.sync_copy(acc_sh.at[pl.ds(s * R0, R_LAST)],
                        out_hbm.at[pl.ds(c * N + s * R0, R_LAST)])


_scatter_call = functools.partial(
    pl.kernel, _scatter_body,
    out_type=jax.ShapeDtypeStruct((NC * N, D_IN), jnp.float32),
    mesh=_mesh,
    compiler_params=pltpu.CompilerParams(needs_layout_passes=False),
    scratch_types=[
        pltpu.VMEM((OCT, K), jnp.int32),
        pltpu.VMEM((OCT, K), jnp.int32),
        pltpu.VMEM((OCT, K), jnp.int32),
        pltpu.VMEM((OCT, K), jnp.int32),
        pltpu.VMEM((K, D_IN), jnp.float32),
        pltpu.VMEM((K, D_IN), jnp.float32),
        pltpu.VMEM_SHARED((NA, D_IN), jnp.float32),
        pltpu.SemaphoreType.DMA,
        pltpu.SemaphoreType.DMA,
    ],
)()


_MB = 2000


def _stage1_body(x_ref, w_ref, dall_ref, y_ref, dinv_ref):
    dsum = jnp.sum(dall_ref[...], axis=1) + 1.0
    dv = lax.rsqrt(dsum)[:, None]
    dinv_ref[...] = dv
    xw = jnp.dot(x_ref[...], w_ref[...], preferred_element_type=jnp.float32)
    y_ref[...] = xw * dv


def _stage1(x, W1, d_all):
    return pl.pallas_call(
        _stage1_body,
        grid=(N // _MB, NC),
        in_specs=[
            pl.BlockSpec((_MB, D_IN), lambda i, c: (i, 0)),
            pl.BlockSpec((D_IN, D_IN), lambda i, c: (0, c)),
            pl.BlockSpec((_MB, _NW), lambda i, c: (i, 0)),
        ],
        out_specs=[
            pl.BlockSpec((_MB, D_IN), lambda i, c: (c * (N // _MB) + i, 0)),
            pl.BlockSpec((_MB, 1), lambda i, c: (i, 0)),
        ],
        out_shape=[
            jax.ShapeDtypeStruct((NC * N, D_IN), jnp.float32),
            jax.ShapeDtypeStruct((N, 1), jnp.float32),
        ],
    )(x, W1, d_all)


def _stage3_body(s1a_ref, s1b_ref, dv_ref, b1_ref, w2_ref, y_ref):
    dv = dv_ref[...]
    h0 = s1a_ref[...] * dv + b1_ref[0, :D_IN]
    h1 = s1b_ref[...] * dv + b1_ref[0, D_IN:]
    xw = (jnp.dot(h0, w2_ref[:D_IN, :], preferred_element_type=jnp.float32)
          + jnp.dot(h1, w2_ref[D_IN:, :], preferred_element_type=jnp.float32))
    y_ref[...] = xw * dv


def _stage3(S1, dinv, b1, W2):
    nb = N // _MB
    return pl.pallas_call(
        _stage3_body,
        grid=(nb, NC),
        in_specs=[
            pl.BlockSpec((_MB, D_IN), lambda i, c: (i, 0)),
            pl.BlockSpec((_MB, D_IN), lambda i, c: (nb + i, 0)),
            pl.BlockSpec((_MB, 1), lambda i, c: (i, 0)),
            pl.BlockSpec((1, D_H), lambda i, c: (0, 0)),
            pl.BlockSpec((D_H, D_IN), lambda i, c: (0, c)),
        ],
        out_specs=pl.BlockSpec((_MB, D_IN), lambda i, c: (c * nb + i, 0)),
        out_shape=jax.ShapeDtypeStruct((NC * N, D_IN), jnp.float32),
    )(S1, S1, dinv, b1, W2)


def _stage5_body(s2_ref, dv_ref, b2_ref, out_ref):
    out_ref[...] = s2_ref[...] * dv_ref[...] + b2_ref[0, :]


def _stage5(S2, dinv, b2):
    nb = N // _MB
    return pl.pallas_call(
        _stage5_body,
        grid=(nb, NC),
        in_specs=[
            pl.BlockSpec((_MB, D_IN), lambda i, c: (c * nb + i, 0)),
            pl.BlockSpec((_MB, 1), lambda i, c: (i, 0)),
            pl.BlockSpec((1, D_IN), lambda i, c: (0, c)),
        ],
        out_specs=pl.BlockSpec((_MB, D_IN), lambda i, c: (i, c)),
        out_shape=jax.ShapeDtypeStruct((N, D_H), jnp.float32),
    )(S2, dinv, b2)


def kernel(inputs, edge_index, W1, b1, W2, b2):
    row = edge_index[0]
    col = edge_index[1]
    pad = E_PAD - E
    rowp = jnp.concatenate([row, jnp.zeros((pad,), jnp.int32)])
    colp = jnp.concatenate([col, jnp.full((pad,), N, jnp.int32)])
    gidx = jnp.concatenate([rowp, rowp + N]).reshape(2 * E_PAD // K, K)
    cidx = colp.reshape(E_PAD // K, K)

    d_all = _deg_call(colp).reshape(_NW, N_H)[:, :N].T

    y1, dinv = _stage1(inputs, W1, d_all)
    S1 = _scatter_call(y1, gidx, cidx)
    y2 = _stage3(S1, dinv, b1.reshape(1, D_H), W2)
    S2 = _scatter_call(y2, gidx, cidx)
    return _stage5(S2, dinv, b2.reshape(1, D_H))


# speedup vs baseline: 1.3416x; 1.3416x over previous
import functools

import jax
import jax.numpy as jnp
from jax import lax
from jax.experimental import pallas as pl
from jax.experimental.pallas import tpu as pltpu
from jax.experimental.pallas import tpu_sc as plsc

N = 10000
E = 320000
D_IN = 128
D_H = 256
NC, NS = 2, 16
K = 128

E_PT = ((E // NS + K - 1) // K) * K
E_PAD = E_PT * NS
N_H = 10240
NA = N + 16
R0 = 632
R_LAST = N - (NS - 1) * R0

_mesh = plsc.VectorSubcoreMesh(
    core_axis_name="c", subcore_axis_name="s", num_cores=NC, num_subcores=NS)


_HR = N_H // 16
_NW = NC * NS


def _deg_body(col_hbm, deg_hbm, col_v, hist):
    c = lax.axis_index("c")
    s = lax.axis_index("s")
    w = c * NS + s
    e_pw = E_PAD // _NW
    z16 = jnp.zeros((16,), jnp.float32)

    def zero_hist(i, _):
        hist[i, :] = z16
        return 0
    lax.fori_loop(0, _HR, zero_hist, 0)

    pltpu.sync_copy(col_hbm.at[pl.ds(w * e_pw, e_pw)], col_v)
    ones = jnp.ones((16,), jnp.float32)

    def count(i, _):
        idx = col_v[pl.ds(i * 16, 16)]
        plsc.addupdate_scatter(hist, [idx >> 4, idx & 15], ones)
        return 0
    lax.fori_loop(0, e_pw // 16, count, 0)

    pltpu.sync_copy(hist, deg_hbm.at[pl.ds(w * _HR, _HR)])


_deg_call = functools.partial(
    pl.kernel, _deg_body,
    out_type=jax.ShapeDtypeStruct((_NW * _HR, 16), jnp.float32),
    mesh=_mesh,
    compiler_params=pltpu.CompilerParams(needs_layout_passes=False),
    scratch_types=[
        pltpu.VMEM((E_PAD // (NC * NS),), jnp.int32),
        pltpu.VMEM((_HR, 16), jnp.float32),
    ],
)()


def _scatter_body(y_hbm, row_hbm, col_hbm, out_hbm, idxg, idxc, rows_v,
                  acc_sh, sem):
    c = lax.axis_index("c")
    s = lax.axis_index("s")
    @pl.when(s < NS - 1)
    def _init_main():
        pltpu.sync_copy(y_hbm.at[pl.ds(c * N + s * R0, R0)],
                        acc_sh.at[pl.ds(s * R0, R0)])

    @pl.when(s == NS - 1)
    def _init_last():
        pltpu.sync_copy(y_hbm.at[pl.ds(c * N + s * R0, R_LAST)],
                        acc_sh.at[pl.ds(s * R0, R_LAST)])
    plsc.subcore_barrier()

    e0 = s * E_PT

    def step(i, _):
        base = e0 + i * K
        pltpu.sync_copy(row_hbm.at[pl.ds(c * E_PAD + base, K)], idxg)
        pltpu.sync_copy(col_hbm.at[pl.ds(base, K)], idxc)
        pltpu.async_copy(y_hbm.at[idxg], rows_v, sem).wait()
        pltpu.sync_copy(rows_v, acc_sh.at[idxc], add=True)
        return 0
    lax.fori_loop(0, E_PT // K, step, 0)

    plsc.subcore_barrier()

    @pl.when(s < NS - 1)
    def _out_main():
        pltpu.sync_copy(acc_sh.at[pl.ds(s * R0, R0)],
                        out_hbm.at[pl.ds(c * N + s * R0, R0)])

    @pl.when(s == NS - 1)
    def _out_last():
        pltpu.sync_copy(acc_sh.at[pl.ds(s * R0, R_LAST)],
                        out_hbm.at[pl.ds(c * N + s * R0, R_LAST)])


_scatter_call = functools.partial(
    pl.kernel, _scatter_body,
    out_type=jax.ShapeDtypeStruct((NC * N, D_IN), jnp.float32),
    mesh=_mesh,
    compiler_params=pltpu.CompilerParams(needs_layout_passes=False),
    scratch_types=[
        pltpu.VMEM((K,), jnp.int32),
        pltpu.VMEM((K,), jnp.int32),
        pltpu.VMEM((K, D_IN), jnp.float32),
        pltpu.VMEM_SHARED((NA, D_IN), jnp.float32),
        pltpu.SemaphoreType.DMA,
    ],
)()


_MB = 2000


def _stage1_body(x_ref, w_ref, dall_ref, y_ref, dinv_ref):
    dsum = jnp.sum(dall_ref[...], axis=1) + 1.0
    dv = lax.rsqrt(dsum)[:, None]
    dinv_ref[...] = dv
    xw = jnp.dot(x_ref[...], w_ref[...], preferred_element_type=jnp.float32)
    y_ref[...] = xw * dv


def _stage1(x, W1, d_all):
    return pl.pallas_call(
        _stage1_body,
        grid=(N // _MB, NC),
        in_specs=[
            pl.BlockSpec((_MB, D_IN), lambda i, c: (i, 0)),
            pl.BlockSpec((D_IN, D_IN), lambda i, c: (0, c)),
            pl.BlockSpec((_MB, _NW), lambda i, c: (i, 0)),
        ],
        out_specs=[
            pl.BlockSpec((_MB, D_IN), lambda i, c: (c * (N // _MB) + i, 0)),
            pl.BlockSpec((_MB, 1), lambda i, c: (i, 0)),
        ],
        out_shape=[
            jax.ShapeDtypeStruct((NC * N, D_IN), jnp.float32),
            jax.ShapeDtypeStruct((N, 1), jnp.float32),
        ],
    )(x, W1, d_all)


def _stage3_body(s1a_ref, s1b_ref, dv_ref, b1_ref, w2_ref, y_ref):
    dv = dv_ref[...]
    h0 = s1a_ref[...] * dv + b1_ref[0, :D_IN]
    h1 = s1b_ref[...] * dv + b1_ref[0, D_IN:]
    xw = (jnp.dot(h0, w2_ref[:D_IN, :], preferred_element_type=jnp.float32)
          + jnp.dot(h1, w2_ref[D_IN:, :], preferred_element_type=jnp.float32))
    y_ref[...] = xw * dv


def _stage3(S1, dinv, b1, W2):
    nb = N // _MB
    return pl.pallas_call(
        _stage3_body,
        grid=(nb, NC),
        in_specs=[
            pl.BlockSpec((_MB, D_IN), lambda i, c: (i, 0)),
            pl.BlockSpec((_MB, D_IN), lambda i, c: (nb + i, 0)),
            pl.BlockSpec((_MB, 1), lambda i, c: (i, 0)),
            pl.BlockSpec((1, D_H), lambda i, c: (0, 0)),
            pl.BlockSpec((D_H, D_IN), lambda i, c: (0, c)),
        ],
        out_specs=pl.BlockSpec((_MB, D_IN), lambda i, c: (c * nb + i, 0)),
        out_shape=jax.ShapeDtypeStruct((NC * N, D_IN), jnp.float32),
    )(S1, S1, dinv, b1, W2)


def _stage5_body(s2_ref, dv_ref, b2_ref, out_ref):
    out_ref[...] = s2_ref[...] * dv_ref[...] + b2_ref[0, :]


def _stage5(S2, dinv, b2):
    nb = N // _MB
    return pl.pallas_call(
        _stage5_body,
        grid=(nb, NC),
        in_specs=[
            pl.BlockSpec((_MB, D_IN), lambda i, c: (c * nb + i, 0)),
            pl.BlockSpec((_MB, 1), lambda i, c: (i, 0)),
            pl.BlockSpec((1, D_IN), lambda i, c: (0, c)),
        ],
        out_specs=pl.BlockSpec((_MB, D_IN), lambda i, c: (i, c)),
        out_shape=jax.ShapeDtypeStruct((N, D_H), jnp.float32),
    )(S2, dinv, b2)


def kernel(inputs, edge_index, W1, b1, W2, b2):
    row = edge_index[0]
    col = edge_index[1]
    pad = E_PAD - E
    rowp = jnp.concatenate([row, jnp.zeros((pad,), jnp.int32)])
    colp = jnp.concatenate([col, jnp.full((pad,), N, jnp.int32)])
    rowcat = jnp.concatenate([rowp, rowp + N])

    d_all = _deg_call(colp).reshape(_NW, N_H)[:, :N].T

    y1, dinv = _stage1(inputs, W1, d_all)
    S1 = _scatter_call(y1, rowcat, colp)
    y2 = _stage3(S1, dinv, b1.reshape(1, D_H), W2)
    S2 = _scatter_call(y2, rowcat, colp)
    return _stage5(S2, dinv, b2.reshape(1, D_H))
